# Initial kernel scaffold; baseline (speedup 1.0000x reference)
import jax
import jax.numpy as jnp
from jax.experimental import pallas as pl


def kernel(hidden_states, ln1_g, ln1_b, Wq, Wk, Wv, Wo, ls_attn, ln2_g, ln2_b, W1, b1, W2, b2, ls_ffn, lnm_g, lnm_b, Wr, We1, be1, We2, be2):
    raise NotImplementedError("write your pallas kernel here")


# speedup vs baseline: 1.5429x; 1.5429x over previous
import functools

import jax
import jax.numpy as jnp
from jax import lax
from jax.experimental import pallas as pl
from jax.experimental.pallas import tpu as pltpu
from jax.experimental.pallas import tpu_sc as plsc

_S = 2048
_H = 1024
_NH = 16
_DH = 64
_I = 4096
_E = 8
_C = 320
_NW = 32
_GPW = (_E * _C) // _NW
_SPW = (_E * _C) // 16


def _dot(a, b):
    return lax.dot_general(a, b, (((a.ndim - 1,), (0,)), ((), ())),
                           preferred_element_type=jnp.float32)


def _ln(x, g, b, eps=1e-5):
    m = jnp.mean(x, axis=-1, keepdims=True)
    v = jnp.mean((x - m) ** 2, axis=-1, keepdims=True)
    return (x - m) * lax.rsqrt(v + eps) * g + b


def _qkv_body(x_ref, g_ref, b_ref, wq_ref, wk_ref, wv_ref,
              q_ref, k_ref, v_ref):
    h = _ln(x_ref[...], g_ref[...], b_ref[...])
    q_ref[...] = _dot(h, wq_ref[...])
    k_ref[...] = _dot(h, wk_ref[...])
    v_ref[...] = _dot(h, wv_ref[...])


def _qkv_call(x, g, b, wq, wk, wv):
    blk = 256
    return pl.pallas_call(
        _qkv_body,
        grid=(_S // blk,),
        in_specs=[
            pl.BlockSpec((blk, _H), lambda i: (i, 0)),
            pl.BlockSpec((_H,), lambda i: (0,)),
            pl.BlockSpec((_H,), lambda i: (0,)),
            pl.BlockSpec((_H, _H), lambda i: (0, 0)),
            pl.BlockSpec((_H, _H), lambda i: (0, 0)),
            pl.BlockSpec((_H, _H), lambda i: (0, 0)),
        ],
        out_specs=[
            pl.BlockSpec((blk, _H), lambda i: (i, 0)),
            pl.BlockSpec((blk, _H), lambda i: (i, 0)),
            pl.BlockSpec((blk, _H), lambda i: (i, 0)),
        ],
        out_shape=[jax.ShapeDtypeStruct((_S, _H), jnp.float32)] * 3,
    )(x, g, b, wq, wk, wv)


def _attn_body(q_ref, k_ref, v_ref, o_ref):
    scale = 1.0 / (_DH ** 0.5)
    qv = q_ref[...] * scale
    kv = k_ref[...]
    vv = v_ref[...]
    for hh in range(2):
        sl = slice(hh * _DH, (hh + 1) * _DH)
        s = lax.dot_general(qv[:, sl], kv[:, sl],
                            (((1,), (1,)), ((), ())),
                            preferred_element_type=jnp.float32)
        m = jnp.max(s, axis=-1, keepdims=True)
        p = jnp.exp(s - m)
        l = jnp.sum(p, axis=-1, keepdims=True)
        o_ref[:, sl] = _dot(p, vv[:, sl]) / l


def _attn_call(q, k, v):
    qblk = 1024
    return pl.pallas_call(
        _attn_body,
        grid=(_NH // 2, _S // qblk),
        in_specs=[
            pl.BlockSpec((qblk, 2 * _DH), lambda hi, qi: (qi, hi)),
            pl.BlockSpec((_S, 2 * _DH), lambda hi, qi: (0, hi)),
            pl.BlockSpec((_S, 2 * _DH), lambda hi, qi: (0, hi)),
        ],
        out_specs=pl.BlockSpec((qblk, 2 * _DH), lambda hi, qi: (qi, hi)),
        out_shape=jax.ShapeDtypeStruct((_S, _H), jnp.float32),
    )(q, k, v)


def _post_body(x_ref, o_ref, wo_ref, lsa_ref, g2_ref, b2_ref,
               w1_ref, bf1_ref, w2_ref, bf2_ref, lsf_ref,
               gm_ref, bm_ref, wr_ref,
               x3_ref, hm_ref, lg_ref):
    x2 = x_ref[...] + lsa_ref[...] * _dot(o_ref[...], wo_ref[...])
    h2 = _ln(x2, g2_ref[...], b2_ref[...])
    f = _dot(jax.nn.gelu(_dot(h2, w1_ref[...]) + bf1_ref[...]),
             w2_ref[...]) + bf2_ref[...]
    x3 = x2 + lsf_ref[...] * f
    hm = _ln(x3, gm_ref[...], bm_ref[...])
    x3_ref[...] = x3
    hm_ref[...] = hm
    lg_ref[...] = _dot(hm, wr_ref[...])


def _post_call(x, o, wo, lsa, g2, b2, w1, bf1, w2, bf2, lsf, gm, bm, wr_pad):
    blk = 256
    vec = lambda n: pl.BlockSpec((n,), lambda i: (0,))
    return pl.pallas_call(
        _post_body,
        grid=(_S // blk,),
        in_specs=[
            pl.BlockSpec((blk, _H), lambda i: (i, 0)),
            pl.BlockSpec((blk, _H), lambda i: (i, 0)),
            pl.BlockSpec((_H, _H), lambda i: (0, 0)),
            vec(_H), vec(_H), vec(_H),
            pl.BlockSpec((_H, _I), lambda i: (0, 0)),
            vec(_I),
            pl.BlockSpec((_I, _H), lambda i: (0, 0)),
            vec(_H), vec(_H), vec(_H), vec(_H),
            pl.BlockSpec((_H, 128), lambda i: (0, 0)),
        ],
        out_specs=[
            pl.BlockSpec((blk, _H), lambda i: (i, 0)),
            pl.BlockSpec((blk, _H), lambda i: (i, 0)),
            pl.BlockSpec((blk, 128), lambda i: (i, 0)),
        ],
        out_shape=[
            jax.ShapeDtypeStruct((_S, _H), jnp.float32),
            jax.ShapeDtypeStruct((_S, _H), jnp.float32),
            jax.ShapeDtypeStruct((_S, 128), jnp.float32),
        ],
    )(x, o, wo, lsa, g2, b2, w1, bf1, w2, bf2, lsf, gm, bm, wr_pad)


def _cumsum0(x):
    n = x.shape[0]
    k = 1
    while k < n:
        shifted = jnp.concatenate(
            [jnp.zeros((k, x.shape[1]), x.dtype), x[:n - k]], axis=0)
        x = x + shifted
        k *= 2
    return x


def _router_body(lg_ref, idx_ref, gs_ref):
    lg = lg_ref[...][:, :_E]
    m = jnp.max(lg, axis=0, keepdims=True)
    p = jnp.exp(lg - m)
    gates = p / jnp.sum(p, axis=0, keepdims=True)

    bits = lax.bitcast_convert_type(lg, jnp.uint32)
    top = jnp.uint32(0x80000000)
    key = jnp.where(bits < top, bits ^ top, ~bits)

    prefix = jnp.zeros((1, _E), jnp.uint32)
    for bit in range(31, -1, -1):
        cand = prefix | jnp.uint32(1 << bit)
        cnt = jnp.sum((key >= cand).astype(jnp.int32), axis=0, keepdims=True)
        prefix = jnp.where(cnt >= _C, cand, prefix)
    theta = prefix

    gt = key > theta
    cnt_gt = jnp.sum(gt.astype(jnp.int32), axis=0, keepdims=True)
    eqm = key == theta
    eq_rank = _cumsum0(eqm.astype(jnp.float32)) - eqm.astype(jnp.float32)
    need = (_C - cnt_gt).astype(jnp.float32)
    sel = gt | (eqm & (eq_rank < need))
    pos = _cumsum0(sel.astype(jnp.float32))

    tio = lax.broadcasted_iota(jnp.int32, (_S, 1), 0).astype(jnp.float32)
    cio = lax.broadcasted_iota(jnp.int32, (1, _C), 1).astype(jnp.float32)
    for e in range(_E):
        pe = pos[:, e:e + 1]
        se = sel[:, e:e + 1]
        ge = gates[:, e:e + 1]
        ind = ((pe == cio + 1.0) & se).astype(jnp.float32)
        idx_ref[e:e + 1, :] = jnp.sum(ind * tio, axis=0,
                                      keepdims=True).astype(jnp.int32)
        gs_ref[e:e + 1, :] = jnp.sum(ind * ge, axis=0, keepdims=True)


def _router_call(lg_pad):
    return pl.pallas_call(
        _router_body,
        in_specs=[pl.BlockSpec((_S, 128), lambda: (0, 0))],
        out_specs=[
            pl.BlockSpec((_E, _C), lambda: (0, 0)),
            pl.BlockSpec((_E, _C), lambda: (0, 0)),
        ],
        out_shape=[
            jax.ShapeDtypeStruct((_E, _C), jnp.int32),
            jax.ShapeDtypeStruct((_E, _C), jnp.float32),
        ],
    )(lg_pad)


@functools.lru_cache(maxsize=None)
def _sc_mesh():
    return plsc.VectorSubcoreMesh(core_axis_name="c", subcore_axis_name="s")


@functools.lru_cache(maxsize=None)
def _sc_gather_kernel():
    @functools.partial(
        pl.kernel,
        mesh=_sc_mesh(),
        out_type=jax.ShapeDtypeStruct((_E * _C, _H), jnp.float32),
        scratch_types=[
            pltpu.VMEM((_GPW,), jnp.int32),
            pltpu.VMEM((_GPW, _H), jnp.float32),
            pltpu.SemaphoreType.DMA,
        ],
    )
    def _sc_gather(hm_hbm, idx_hbm, out_hbm, idx_v, rows_v, sem):
        wid = lax.axis_index("s") * 2 + lax.axis_index("c")
        base = wid * _GPW
        pltpu.sync_copy(idx_hbm.at[pl.ds(base, _GPW)], idx_v)
        pltpu.async_copy(hm_hbm.at[idx_v], rows_v, sem).wait()
        pltpu.sync_copy(rows_v, out_hbm.at[pl.ds(base, _GPW)])

    return _sc_gather


def _expert_body(tok_ref, w1_ref, b1_ref, w2_ref, b2_ref, gs_ref, y_ref):
    i = pl.program_id(1)
    a = jax.nn.gelu(_dot(tok_ref[...], w1_ref[0]) + b1_ref[0])
    part = _dot(a, w2_ref[0])

    @pl.when(i == 0)
    def _():
        y_ref[...] = part

    @pl.when(i == 1)
    def _():
        gcol = jnp.transpose(gs_ref[0], (1, 0))
        y_ref[...] = (y_ref[...] + part + b2_ref[0]) * gcol


def _expert_call(tok, we1, be1r, we2, be2r, gsr):
    iblk = _I // 2
    return pl.pallas_call(
        _expert_body,
        grid=(_E, 2),
        in_specs=[
            pl.BlockSpec((_C, _H), lambda e, i: (e, 0)),
            pl.BlockSpec((1, _H, iblk), lambda e, i: (e, 0, i)),
            pl.BlockSpec((1, 1, iblk), lambda e, i: (e, 0, i)),
            pl.BlockSpec((1, iblk, _H), lambda e, i: (e, i, 0)),
            pl.BlockSpec((1, 1, _H), lambda e, i: (e, 0, 0)),
            pl.BlockSpec((1, 1, _C), lambda e, i: (e, 0, 0)),
        ],
        out_specs=pl.BlockSpec((_C, _H), lambda e, i: (e, 0)),
        out_shape=jax.ShapeDtypeStruct((_E * _C, _H), jnp.float32),
    )(tok, we1, be1r, we2, be2r, gsr)


def _combine_body(idxf_ref, y_ref, x3_ref, out_ref):
    blk = out_ref.shape[0]
    t0 = pl.program_id(0) * blk
    tcol = (lax.broadcasted_iota(jnp.int32, (blk, 1), 0) + t0).astype(
        jnp.float32)
    idxr = idxf_ref[...].reshape(1, _E * _C)
    onehot = (tcol == idxr).astype(jnp.float32)
    out_ref[...] = x3_ref[...] + _dot(onehot, y_ref[...])


def _combine_call(idx_f, y, x3):
    blk = 256
    return pl.pallas_call(
        _combine_body,
        grid=(_S // blk,),
        in_specs=[
            pl.BlockSpec((_E * _C,), lambda i: (0,)),
            pl.BlockSpec((_E * _C, _H), lambda i: (0, 0)),
            pl.BlockSpec((blk, _H), lambda i: (i, 0)),
        ],
        out_specs=pl.BlockSpec((blk, _H), lambda i: (i, 0)),
        out_shape=jax.ShapeDtypeStruct((_S, _H), jnp.float32),
    )(idx_f, y, x3)


def kernel(hidden_states, ln1_g, ln1_b, Wq, Wk, Wv, Wo, ls_attn, ln2_g,
           ln2_b, W1, b1, W2, b2, ls_ffn, lnm_g, lnm_b, Wr, We1, be1,
           We2, be2):
    x = hidden_states.reshape(_S, _H)
    q, k, v = _qkv_call(x, ln1_g, ln1_b, Wq, Wk, Wv)
    o = _attn_call(q, k, v)
    wr_pad = jnp.pad(Wr, ((0, 0), (0, 128 - _E)))
    x3, hm, lg_pad = _post_call(x, o, Wo, ls_attn, ln2_g, ln2_b, W1, b1,
                                W2, b2, ls_ffn, lnm_g, lnm_b, wr_pad)
    logits = lg_pad[:, :_E]
    idx, gsel = _router_call(lg_pad)
    idx_flat = idx.reshape(-1)
    tok = _sc_gather_kernel()(hm, idx_flat)
    y = _expert_call(tok, We1, be1.reshape(_E, 1, _I), We2,
                     be2.reshape(_E, 1, _H), gsel.reshape(_E, 1, _C))
    out = _combine_call(idx_flat.astype(jnp.float32), y, x3)
    return out.reshape(1, _S, _H), logits


# speedup vs baseline: 1.7021x; 1.1032x over previous
import functools

import jax
import jax.numpy as jnp
from jax import lax
from jax.experimental import pallas as pl
from jax.experimental.pallas import tpu as pltpu
from jax.experimental.pallas import tpu_sc as plsc

_S = 2048
_H = 1024
_NH = 16
_DH = 64
_I = 4096
_E = 8
_C = 320
_NW = 32
_GPW = (_E * _C) // _NW
_SPW = (_E * _C) // 16


def _dot(a, b):
    return lax.dot_general(a, b, (((a.ndim - 1,), (0,)), ((), ())),
                           preferred_element_type=jnp.float32)


def _ln(x, g, b, eps=1e-5):
    m = jnp.mean(x, axis=-1, keepdims=True)
    v = jnp.mean((x - m) ** 2, axis=-1, keepdims=True)
    return (x - m) * lax.rsqrt(v + eps) * g + b


def _qkv_body(x_ref, g_ref, b_ref, wq_ref, wk_ref, wv_ref,
              q_ref, k_ref, v_ref):
    h = _ln(x_ref[...], g_ref[...], b_ref[...])
    q_ref[...] = _dot(h, wq_ref[...])
    k_ref[...] = _dot(h, wk_ref[...])
    v_ref[...] = _dot(h, wv_ref[...])


def _qkv_call(x, g, b, wq, wk, wv):
    blk = 256
    return pl.pallas_call(
        _qkv_body,
        grid=(_S // blk,),
        in_specs=[
            pl.BlockSpec((blk, _H), lambda i: (i, 0)),
            pl.BlockSpec((_H,), lambda i: (0,)),
            pl.BlockSpec((_H,), lambda i: (0,)),
            pl.BlockSpec((_H, _H), lambda i: (0, 0)),
            pl.BlockSpec((_H, _H), lambda i: (0, 0)),
            pl.BlockSpec((_H, _H), lambda i: (0, 0)),
        ],
        out_specs=[
            pl.BlockSpec((blk, _H), lambda i: (i, 0)),
            pl.BlockSpec((blk, _H), lambda i: (i, 0)),
            pl.BlockSpec((blk, _H), lambda i: (i, 0)),
        ],
        out_shape=[jax.ShapeDtypeStruct((_S, _H), jnp.float32)] * 3,
    )(x, g, b, wq, wk, wv)


def _attn_body(q_ref, k_ref, v_ref, o_ref):
    scale = 1.0 / (_DH ** 0.5)
    qv = (q_ref[...] * scale).astype(jnp.bfloat16)
    kv = k_ref[...].astype(jnp.bfloat16)
    vv = v_ref[...].astype(jnp.bfloat16)
    for hh in range(2):
        sl = slice(hh * _DH, (hh + 1) * _DH)
        s = lax.dot_general(qv[:, sl], kv[:, sl],
                            (((1,), (1,)), ((), ())),
                            preferred_element_type=jnp.float32)
        p = jnp.exp(s)
        l = jnp.sum(p, axis=-1, keepdims=True)
        o_ref[:, sl] = _dot(p.astype(jnp.bfloat16), vv[:, sl]) / l


def _attn_call(q, k, v):
    qblk = 1024
    return pl.pallas_call(
        _attn_body,
        grid=(_NH // 2, _S // qblk),
        in_specs=[
            pl.BlockSpec((qblk, 2 * _DH), lambda hi, qi: (qi, hi)),
            pl.BlockSpec((_S, 2 * _DH), lambda hi, qi: (0, hi)),
            pl.BlockSpec((_S, 2 * _DH), lambda hi, qi: (0, hi)),
        ],
        out_specs=pl.BlockSpec((qblk, 2 * _DH), lambda hi, qi: (qi, hi)),
        out_shape=jax.ShapeDtypeStruct((_S, _H), jnp.float32),
    )(q, k, v)


def _post_body(x_ref, o_ref, wo_ref, lsa_ref, g2_ref, b2_ref,
               w1_ref, bf1_ref, w2_ref, bf2_ref, lsf_ref,
               gm_ref, bm_ref, wr_ref,
               x3_ref, hm_ref, lg_ref):
    x2 = x_ref[...] + lsa_ref[...] * _dot(o_ref[...], wo_ref[...])
    h2 = _ln(x2, g2_ref[...], b2_ref[...])
    f = _dot(jax.nn.gelu(_dot(h2, w1_ref[...]) + bf1_ref[...]),
             w2_ref[...]) + bf2_ref[...]
    x3 = x2 + lsf_ref[...] * f
    hm = _ln(x3, gm_ref[...], bm_ref[...])
    x3_ref[...] = x3
    hm_ref[...] = hm
    lg_ref[...] = _dot(hm, wr_ref[...])


def _post_call(x, o, wo, lsa, g2, b2, w1, bf1, w2, bf2, lsf, gm, bm, wr_pad):
    blk = 256
    vec = lambda n: pl.BlockSpec((n,), lambda i: (0,))
    return pl.pallas_call(
        _post_body,
        grid=(_S // blk,),
        in_specs=[
            pl.BlockSpec((blk, _H), lambda i: (i, 0)),
            pl.BlockSpec((blk, _H), lambda i: (i, 0)),
            pl.BlockSpec((_H, _H), lambda i: (0, 0)),
            vec(_H), vec(_H), vec(_H),
            pl.BlockSpec((_H, _I), lambda i: (0, 0)),
            vec(_I),
            pl.BlockSpec((_I, _H), lambda i: (0, 0)),
            vec(_H), vec(_H), vec(_H), vec(_H),
            pl.BlockSpec((_H, 128), lambda i: (0, 0)),
        ],
        out_specs=[
            pl.BlockSpec((blk, _H), lambda i: (i, 0)),
            pl.BlockSpec((blk, _H), lambda i: (i, 0)),
            pl.BlockSpec((blk, 128), lambda i: (i, 0)),
        ],
        out_shape=[
            jax.ShapeDtypeStruct((_S, _H), jnp.float32),
            jax.ShapeDtypeStruct((_S, _H), jnp.float32),
            jax.ShapeDtypeStruct((_S, 128), jnp.float32),
        ],
    )(x, o, wo, lsa, g2, b2, w1, bf1, w2, bf2, lsf, gm, bm, wr_pad)


def _cumsum0(x):
    n = x.shape[0]
    k = 1
    while k < n:
        shifted = jnp.concatenate(
            [jnp.zeros((k, x.shape[1]), x.dtype), x[:n - k]], axis=0)
        x = x + shifted
        k *= 2
    return x


def _router_body(lg_ref, idx_ref, gs_ref):
    lg = lg_ref[...][:, :_E]
    m = jnp.max(lg, axis=0, keepdims=True)
    p = jnp.exp(lg - m)
    gates = p / jnp.sum(p, axis=0, keepdims=True)

    bits = lax.bitcast_convert_type(lg, jnp.uint32)
    top = jnp.uint32(0x80000000)
    key = jnp.where(bits < top, bits ^ top, ~bits)

    prefix = jnp.zeros((1, _E), jnp.uint32)
    for bit in range(31, -1, -1):
        cand = prefix | jnp.uint32(1 << bit)
        cnt = jnp.sum((key >= cand).astype(jnp.int32), axis=0, keepdims=True)
        prefix = jnp.where(cnt >= _C, cand, prefix)
    theta = prefix

    gt = key > theta
    cnt_gt = jnp.sum(gt.astype(jnp.int32), axis=0, keepdims=True)
    eqm = key == theta
    eq_rank = _cumsum0(eqm.astype(jnp.float32)) - eqm.astype(jnp.float32)
    need = (_C - cnt_gt).astype(jnp.float32)
    sel = gt | (eqm & (eq_rank < need))
    pos = _cumsum0(sel.astype(jnp.float32))

    tio = lax.broadcasted_iota(jnp.int32, (_S, 1), 0).astype(jnp.float32)
    cio = lax.broadcasted_iota(jnp.int32, (1, _C), 1).astype(jnp.float32)
    for e in range(_E):
        pe = pos[:, e:e + 1]
        se = sel[:, e:e + 1]
        ge = gates[:, e:e + 1]
        ind = ((pe == cio + 1.0) & se).astype(jnp.float32)
        idx_ref[e:e + 1, :] = jnp.sum(ind * tio, axis=0,
                                      keepdims=True).astype(jnp.int32)
        gs_ref[e:e + 1, :] = jnp.sum(ind * ge, axis=0, keepdims=True)


def _router_call(lg_pad):
    return pl.pallas_call(
        _router_body,
        in_specs=[pl.BlockSpec((_S, 128), lambda: (0, 0))],
        out_specs=[
            pl.BlockSpec((_E, _C), lambda: (0, 0)),
            pl.BlockSpec((_E, _C), lambda: (0, 0)),
        ],
        out_shape=[
            jax.ShapeDtypeStruct((_E, _C), jnp.int32),
            jax.ShapeDtypeStruct((_E, _C), jnp.float32),
        ],
    )(lg_pad)


@functools.lru_cache(maxsize=None)
def _sc_mesh():
    return plsc.VectorSubcoreMesh(core_axis_name="c", subcore_axis_name="s")


@functools.lru_cache(maxsize=None)
def _sc_gather_kernel():
    @functools.partial(
        pl.kernel,
        mesh=_sc_mesh(),
        out_type=jax.ShapeDtypeStruct((_E * _C, _H), jnp.float32),
        scratch_types=[
            pltpu.VMEM((_GPW,), jnp.int32),
            pltpu.VMEM((_GPW, _H), jnp.float32),
            pltpu.SemaphoreType.DMA,
        ],
    )
    def _sc_gather(hm_hbm, idx_hbm, out_hbm, idx_v, rows_v, sem):
        wid = lax.axis_index("s") * 2 + lax.axis_index("c")
        base = wid * _GPW
        pltpu.sync_copy(idx_hbm.at[pl.ds(base, _GPW)], idx_v)
        pltpu.async_copy(hm_hbm.at[idx_v], rows_v, sem).wait()
        pltpu.sync_copy(rows_v, out_hbm.at[pl.ds(base, _GPW)])

    return _sc_gather


def _expert_body(tok_ref, w1_ref, b1_ref, w2_ref, b2_ref, gs_ref, y_ref,
                 acc_ref):
    i = pl.program_id(1)
    a = jax.nn.gelu(_dot(tok_ref[...], w1_ref[0]) + b1_ref[0])
    part = _dot(a, w2_ref[0])

    @pl.when(i == 0)
    def _():
        acc_ref[...] = part

    @pl.when(i == 1)
    def _():
        gcol = jnp.transpose(gs_ref[0], (1, 0))
        y_ref[...] = ((acc_ref[...] + part + b2_ref[0]) * gcol).astype(
            jnp.bfloat16)


def _expert_call(tok, we1, be1r, we2, be2r, gsr):
    iblk = _I // 2
    return pl.pallas_call(
        _expert_body,
        grid=(_E, 2),
        in_specs=[
            pl.BlockSpec((_C, _H), lambda e, i: (e, 0)),
            pl.BlockSpec((1, _H, iblk), lambda e, i: (e, 0, i)),
            pl.BlockSpec((1, 1, iblk), lambda e, i: (e, 0, i)),
            pl.BlockSpec((1, iblk, _H), lambda e, i: (e, i, 0)),
            pl.BlockSpec((1, 1, _H), lambda e, i: (e, 0, 0)),
            pl.BlockSpec((1, 1, _C), lambda e, i: (e, 0, 0)),
        ],
        out_specs=pl.BlockSpec((_C, _H), lambda e, i: (e, 0)),
        out_shape=jax.ShapeDtypeStruct((_E * _C, _H), jnp.bfloat16),
        scratch_shapes=[pltpu.VMEM((_C, _H), jnp.float32)],
    )(tok, we1, be1r, we2, be2r, gsr)


def _combine_body(idxf_ref, y_ref, x3_ref, out_ref):
    blk = out_ref.shape[0]
    t0 = pl.program_id(0) * blk
    tcol = (lax.broadcasted_iota(jnp.int32, (blk, 1), 0) + t0).astype(
        jnp.float32)
    idxr = idxf_ref[...].reshape(1, _E * _C)
    onehot = (tcol == idxr).astype(jnp.bfloat16)
    out_ref[...] = x3_ref[...] + _dot(onehot, y_ref[...])


def _combine_call(idx_f, y, x3):
    blk = 256
    return pl.pallas_call(
        _combine_body,
        grid=(_S // blk,),
        in_specs=[
            pl.BlockSpec((_E * _C,), lambda i: (0,)),
            pl.BlockSpec((_E * _C, _H), lambda i: (0, 0)),
            pl.BlockSpec((blk, _H), lambda i: (i, 0)),
        ],
        out_specs=pl.BlockSpec((blk, _H), lambda i: (i, 0)),
        out_shape=jax.ShapeDtypeStruct((_S, _H), jnp.float32),
    )(idx_f, y, x3)


def kernel(hidden_states, ln1_g, ln1_b, Wq, Wk, Wv, Wo, ls_attn, ln2_g,
           ln2_b, W1, b1, W2, b2, ls_ffn, lnm_g, lnm_b, Wr, We1, be1,
           We2, be2):
    x = hidden_states.reshape(_S, _H)
    q, k, v = _qkv_call(x, ln1_g, ln1_b, Wq, Wk, Wv)
    o = _attn_call(q, k, v)
    wr_pad = jnp.pad(Wr, ((0, 0), (0, 128 - _E)))
    x3, hm, lg_pad = _post_call(x, o, Wo, ls_attn, ln2_g, ln2_b, W1, b1,
                                W2, b2, ls_ffn, lnm_g, lnm_b, wr_pad)
    logits = lg_pad[:, :_E]
    idx, gsel = _router_call(lg_pad)
    idx_flat = idx.reshape(-1)
    tok = _sc_gather_kernel()(hm, idx_flat)
    y = _expert_call(tok, We1, be1.reshape(_E, 1, _I), We2,
                     be2.reshape(_E, 1, _H), gsel.reshape(_E, 1, _C))
    out = _combine_call(idx_flat.astype(jnp.float32), y, x3)
    return out.reshape(1, _S, _H), logits
